# Initial kernel scaffold; baseline (speedup 1.0000x reference)
import jax
import jax.numpy as jnp
from jax.experimental import pallas as pl


def kernel(x, edge_index, batch, eps0, W0a, b0a, W0b, b0b, eps1, W1a, b1a, W1b, b1b, eps2, W2a, b2a, W2b, b2b, Wf1, bf1, Wf2, bf2):
    raise NotImplementedError("write your pallas kernel here")


# speedup vs baseline: 4.3159x; 4.3159x over previous
import functools

import jax
import jax.numpy as jnp
from jax import lax
from jax.experimental import pallas as pl
from jax.experimental.pallas import tpu as pltpu
from jax.experimental.pallas import tpu_sc as plsc

NC = 2
NS = 16
CHUNK = 80


def _scatter_add_factory(n_edges, rows_alloc, zchunk, d):
  edges_per_tile = n_edges // (NC * NS)
  n_chunks = edges_per_tile // CHUNK
  assert n_chunks * CHUNK == edges_per_tile
  zrpt = rows_alloc // NS
  nz = zrpt // zchunk
  assert nz * zchunk == zrpt and zrpt * NS == rows_alloc and zrpt % 8 == 0
  mesh = plsc.VectorSubcoreMesh(
      core_axis_name="c", subcore_axis_name="s",
      num_cores=NC, num_subcores=NS)

  @functools.partial(
      pl.kernel,
      out_type=jax.ShapeDtypeStruct((NC, rows_alloc, d), jnp.float32),
      mesh=mesh,
      scratch_types=[
          pltpu.VMEM((CHUNK,), jnp.int32),
          pltpu.VMEM((CHUNK,), jnp.int32),
          pltpu.VMEM((CHUNK, d), jnp.float32),
          pltpu.VMEM_SHARED((rows_alloc, d), jnp.float32),
          pltpu.SemaphoreType.DMA,
      ],
  )
  def scatter_add(h_hbm, src_hbm, dst_hbm, zero_hbm, out_hbm,
                  sidx, didx, rows, agg, sem):
    c = lax.axis_index("c")
    s = lax.axis_index("s")
    for j in range(nz):
      pltpu.sync_copy(zero_hbm, agg.at[pl.ds(s * zrpt + j * zchunk, zchunk)])
    plsc.subcore_barrier()
    tile_base = (c * NS + s) * edges_per_tile

    def body(k, carry):
      base = tile_base + k * CHUNK
      pltpu.sync_copy(src_hbm.at[pl.ds(base, CHUNK)], sidx)
      pltpu.sync_copy(dst_hbm.at[pl.ds(base, CHUNK)], didx)
      pltpu.async_copy(h_hbm.at[sidx], rows, sem).wait()
      pltpu.sync_copy(rows, agg.at[didx], add=True)
      return carry

    lax.fori_loop(0, n_chunks, body, 0)
    plsc.subcore_barrier()
    pltpu.sync_copy(agg.at[pl.ds(s * zrpt, zrpt)],
                    out_hbm.at[c, pl.ds(s * zrpt, zrpt)])

  return scatter_add


def _mlp_body(scale_ref, x_ref, a0_ref, a1_ref, wa_ref, ba_ref, wb_ref,
              bb_ref, o_ref):
  h = x_ref[...] * scale_ref[0] + a0_ref[...] + a1_ref[...]
  h = jnp.dot(h, wa_ref[...], preferred_element_type=jnp.float32) + ba_ref[...]
  h = jnp.maximum(h, 0.0)
  h = jnp.dot(h, wb_ref[...], preferred_element_type=jnp.float32) + bb_ref[...]
  o_ref[...] = jnp.maximum(h, 0.0)


def _mlp(scale, x, a0, a1, Wa, ba, Wb, bb, block=1024):
  n, d = x.shape
  dh = Wa.shape[1]
  do = Wb.shape[1]
  return pl.pallas_call(
      _mlp_body,
      grid=(n // block,),
      in_specs=[
          pl.BlockSpec(memory_space=pltpu.SMEM),
          pl.BlockSpec((block, d), lambda i: (i, 0)),
          pl.BlockSpec((block, d), lambda i: (i, 0)),
          pl.BlockSpec((block, d), lambda i: (i, 0)),
          pl.BlockSpec((d, dh), lambda i: (0, 0)),
          pl.BlockSpec((1, dh), lambda i: (0, 0)),
          pl.BlockSpec((dh, do), lambda i: (0, 0)),
          pl.BlockSpec((1, do), lambda i: (0, 0)),
      ],
      out_specs=pl.BlockSpec((block, do), lambda i: (i, 0)),
      out_shape=jax.ShapeDtypeStruct((n, do), jnp.float32),
  )(scale, x, a0, a1, Wa, ba, Wb, bb)


def _final_body(p0_ref, p1_ref, w1_ref, b1_ref, w2_ref, b2_ref, o_ref):
  p = p0_ref[...] + p1_ref[...]
  t = jnp.dot(p, w1_ref[...], preferred_element_type=jnp.float32) + b1_ref[...]
  t = jnp.maximum(t, 0.0)
  o_ref[...] = (
      jnp.dot(t, w2_ref[...], preferred_element_type=jnp.float32) + b2_ref[...])


def kernel(x, edge_index, batch, eps0, W0a, b0a, W0b, b0b,
           eps1, W1a, b1a, W1b, b1b, eps2, W2a, b2a, W2b, b2b,
           Wf1, bf1, Wf2, bf2):
  n, d = x.shape
  e = edge_index.shape[1]
  g = 128
  src = edge_index[0]
  dst = edge_index[1]

  tile_quant = NC * NS * CHUNK
  n_pad = ((n + tile_quant - 1) // tile_quant) * tile_quant
  pool_src = jnp.concatenate(
      [jnp.arange(n, dtype=jnp.int32),
       jnp.zeros((n_pad - n,), dtype=jnp.int32)])
  pool_dst = jnp.concatenate(
      [batch.astype(jnp.int32),
       jnp.full((n_pad - n,), g, dtype=jnp.int32)])

  edge_alloc = NS * 128 * (-(-n // (NS * 128)))
  zeros_edge = jnp.zeros((128, d), jnp.float32)
  pool_alloc = 2 * NS * 8
  zeros_pool = jnp.zeros((pool_alloc // NS, d), jnp.float32)

  edge_scat = _scatter_add_factory(e, edge_alloc, 128, d)
  pool_scat = _scatter_add_factory(n_pad, pool_alloc, pool_alloc // NS, d)

  ba0, bb0 = b0a.reshape(1, -1), b0b.reshape(1, -1)
  ba1, bb1 = b1a.reshape(1, -1), b1b.reshape(1, -1)
  ba2, bb2 = b2a.reshape(1, -1), b2b.reshape(1, -1)
  s0 = (1.0 + eps0).reshape(1)
  s1 = (1.0 + eps1).reshape(1)
  s2 = (1.0 + eps2).reshape(1)

  xp = jnp.pad(x, ((0, edge_alloc - n), (0, 0)))
  a = edge_scat(xp, src, dst, zeros_edge)
  h = _mlp(s0, xp, a[0], a[1], W0a, ba0, W0b, bb0)
  a = edge_scat(h, src, dst, zeros_edge)
  h = _mlp(s1, h, a[0], a[1], W1a, ba1, W1b, bb1)
  a = edge_scat(h, src, dst, zeros_edge)
  h = _mlp(s2, h, a[0], a[1], W2a, ba2, W2b, bb2)

  p = pool_scat(h, pool_src, pool_dst, zeros_pool)

  out = pl.pallas_call(
      _final_body,
      out_shape=jax.ShapeDtypeStruct((g, 1), jnp.float32),
  )(p[0, :g], p[1, :g], Wf1, bf1.reshape(1, -1), Wf2, bf2.reshape(1, -1))
  return out


# speedup vs baseline: 9.4091x; 2.1801x over previous
import functools

import jax
import jax.numpy as jnp
from jax import lax
from jax.experimental import pallas as pl
from jax.experimental.pallas import tpu as pltpu
from jax.experimental.pallas import tpu_sc as plsc

NC = 2
NS = 16
CHUNK = 80


def _scatter_add_factory(n_edges, rows_alloc, zchunk, d):
  edges_per_tile = n_edges // (NC * NS)
  n_chunks = edges_per_tile // CHUNK
  assert n_chunks * CHUNK == edges_per_tile
  zrpt = rows_alloc // NS
  nz = zrpt // zchunk
  assert nz * zchunk == zrpt and zrpt * NS == rows_alloc and zrpt % 8 == 0
  mesh = plsc.VectorSubcoreMesh(
      core_axis_name="c", subcore_axis_name="s",
      num_cores=NC, num_subcores=NS)

  npairs = n_chunks // 2
  rem = n_chunks % 2

  @functools.partial(
      pl.kernel,
      out_type=jax.ShapeDtypeStruct((NC, rows_alloc, d), jnp.float32),
      mesh=mesh,
      scratch_types=[
          pltpu.VMEM((edges_per_tile,), jnp.int32),
          pltpu.VMEM((edges_per_tile,), jnp.int32),
          pltpu.VMEM((CHUNK, d), jnp.float32),
          pltpu.VMEM((CHUNK, d), jnp.float32),
          pltpu.VMEM_SHARED((rows_alloc, d), jnp.float32),
          pltpu.SemaphoreType.DMA,
          pltpu.SemaphoreType.DMA,
      ],
  )
  def scatter_add(h_hbm, src_hbm, dst_hbm, zero_hbm, out_hbm,
                  sidx, didx, rows_a, rows_b, agg, sem_a, sem_b):
    c = lax.axis_index("c")
    s = lax.axis_index("s")
    wid = c * NS + s
    pltpu.sync_copy(src_hbm.at[pl.ds(wid * edges_per_tile, edges_per_tile)],
                    sidx)
    pltpu.sync_copy(dst_hbm.at[pl.ds(wid * edges_per_tile, edges_per_tile)],
                    didx)
    for j in range(nz):
      pltpu.sync_copy(zero_hbm, agg.at[pl.ds(s * zrpt + j * zchunk, zchunk)])
    plsc.subcore_barrier()

    ga = pltpu.async_copy(h_hbm.at[sidx.at[pl.ds(0, CHUNK)]], rows_a, sem_a)
    if n_chunks > 1:
      gb = pltpu.async_copy(h_hbm.at[sidx.at[pl.ds(CHUNK, CHUNK)]], rows_b, sem_b)

    def body(k, carry):
      i = 2 * k
      ga.wait()
      pltpu.sync_copy(rows_a, agg.at[didx.at[pl.ds(i * CHUNK, CHUNK)]], add=True)

      @pl.when(i + 2 < n_chunks)
      def _():
        pltpu.async_copy(h_hbm.at[sidx.at[pl.ds((i + 2) * CHUNK, CHUNK)]], rows_a, sem_a)

      gb.wait()
      pltpu.sync_copy(rows_b, agg.at[didx.at[pl.ds((i + 1) * CHUNK, CHUNK)]], add=True)

      @pl.when(i + 3 < n_chunks)
      def _():
        pltpu.async_copy(h_hbm.at[sidx.at[pl.ds((i + 3) * CHUNK, CHUNK)]], rows_b, sem_b)

      return carry

    lax.fori_loop(0, npairs, body, 0)
    if rem:
      ga.wait()
      pltpu.sync_copy(rows_a, agg.at[didx.at[pl.ds((n_chunks - 1) * CHUNK, CHUNK)]], add=True)
    plsc.subcore_barrier()
    pltpu.sync_copy(agg.at[pl.ds(s * zrpt, zrpt)],
                    out_hbm.at[c, pl.ds(s * zrpt, zrpt)])

  return scatter_add


def _mlp_body(scale_ref, x_ref, a0_ref, a1_ref, wa_ref, ba_ref, wb_ref,
              bb_ref, o_ref):
  h = x_ref[...] * scale_ref[0] + a0_ref[...] + a1_ref[...]
  h = jnp.dot(h, wa_ref[...], preferred_element_type=jnp.float32) + ba_ref[...]
  h = jnp.maximum(h, 0.0)
  h = jnp.dot(h, wb_ref[...], preferred_element_type=jnp.float32) + bb_ref[...]
  o_ref[...] = jnp.maximum(h, 0.0)


def _mlp(scale, x, a0, a1, Wa, ba, Wb, bb, block=1024):
  n, d = x.shape
  dh = Wa.shape[1]
  do = Wb.shape[1]
  return pl.pallas_call(
      _mlp_body,
      grid=(n // block,),
      in_specs=[
          pl.BlockSpec(memory_space=pltpu.SMEM),
          pl.BlockSpec((block, d), lambda i: (i, 0)),
          pl.BlockSpec((block, d), lambda i: (i, 0)),
          pl.BlockSpec((block, d), lambda i: (i, 0)),
          pl.BlockSpec((d, dh), lambda i: (0, 0)),
          pl.BlockSpec((1, dh), lambda i: (0, 0)),
          pl.BlockSpec((dh, do), lambda i: (0, 0)),
          pl.BlockSpec((1, do), lambda i: (0, 0)),
      ],
      out_specs=pl.BlockSpec((block, do), lambda i: (i, 0)),
      out_shape=jax.ShapeDtypeStruct((n, do), jnp.float32),
  )(scale, x, a0, a1, Wa, ba, Wb, bb)


def _final_body(p0_ref, p1_ref, w1_ref, b1_ref, w2_ref, b2_ref, o_ref):
  p = p0_ref[...] + p1_ref[...]
  t = jnp.dot(p, w1_ref[...], preferred_element_type=jnp.float32) + b1_ref[...]
  t = jnp.maximum(t, 0.0)
  o_ref[...] = (
      jnp.dot(t, w2_ref[...], preferred_element_type=jnp.float32) + b2_ref[...])


def kernel(x, edge_index, batch, eps0, W0a, b0a, W0b, b0b,
           eps1, W1a, b1a, W1b, b1b, eps2, W2a, b2a, W2b, b2b,
           Wf1, bf1, Wf2, bf2):
  n, d = x.shape
  e = edge_index.shape[1]
  g = 128
  nw = NC * NS
  src = edge_index[0]
  dst = edge_index[1]

  tile_quant = NC * NS * CHUNK
  n_pad = ((n + tile_quant - 1) // tile_quant) * tile_quant
  pool_src = jnp.concatenate(
      [jnp.arange(n, dtype=jnp.int32),
       jnp.zeros((n_pad - n,), dtype=jnp.int32)])
  pool_dst = jnp.concatenate(
      [batch.astype(jnp.int32),
       jnp.full((n_pad - n,), g, dtype=jnp.int32)])

  edge_alloc = NS * 128 * (-(-n // (NS * 128)))
  zeros_edge = jnp.zeros((128, d), jnp.float32)
  pool_alloc = 2 * NS * 8
  zeros_pool = jnp.zeros((pool_alloc // NS, d), jnp.float32)

  edge_scat = _scatter_add_factory(e, edge_alloc, 128, d)
  pool_scat = _scatter_add_factory(n_pad, pool_alloc, pool_alloc // NS, d)

  ba0, bb0 = b0a.reshape(1, -1), b0b.reshape(1, -1)
  ba1, bb1 = b1a.reshape(1, -1), b1b.reshape(1, -1)
  ba2, bb2 = b2a.reshape(1, -1), b2b.reshape(1, -1)
  s0 = (1.0 + eps0).reshape(1)
  s1 = (1.0 + eps1).reshape(1)
  s2 = (1.0 + eps2).reshape(1)

  xp = jnp.pad(x, ((0, edge_alloc - n), (0, 0)))
  a = edge_scat(xp, src, dst, zeros_edge)
  h = _mlp(s0, xp, a[0], a[1], W0a, ba0, W0b, bb0)
  a = edge_scat(h, src, dst, zeros_edge)
  h = _mlp(s1, h, a[0], a[1], W1a, ba1, W1b, bb1)
  a = edge_scat(h, src, dst, zeros_edge)
  h = _mlp(s2, h, a[0], a[1], W2a, ba2, W2b, bb2)

  p = pool_scat(h, pool_src, pool_dst, zeros_pool)

  out = pl.pallas_call(
      _final_body,
      out_shape=jax.ShapeDtypeStruct((g, 1), jnp.float32),
  )(p[0, :g], p[1, :g], Wf1, bf1.reshape(1, -1), Wf2, bf2.reshape(1, -1))
  return out


# speedup vs baseline: 9.9123x; 1.0535x over previous
import functools

import jax
import jax.numpy as jnp
from jax import lax
from jax.experimental import pallas as pl
from jax.experimental.pallas import tpu as pltpu
from jax.experimental.pallas import tpu_sc as plsc

NC = 2
NS = 16
CHUNK = 80


def _scatter_add_factory(n_edges, rows_alloc, zchunk, d):
  edges_per_tile = n_edges // (NC * NS)
  n_chunks = edges_per_tile // CHUNK
  assert n_chunks * CHUNK == edges_per_tile
  zrpt = rows_alloc // NS
  nz = zrpt // zchunk
  assert nz * zchunk == zrpt and zrpt * NS == rows_alloc and zrpt % 8 == 0
  assert zchunk <= CHUNK
  mesh = plsc.VectorSubcoreMesh(
      core_axis_name="c", subcore_axis_name="s",
      num_cores=NC, num_subcores=NS)

  npairs = n_chunks // 2
  rem = n_chunks % 2

  @functools.partial(
      pl.kernel,
      out_type=jax.ShapeDtypeStruct((NC, rows_alloc, d), jnp.float32),
      mesh=mesh,
      scratch_types=[
          pltpu.VMEM((edges_per_tile,), jnp.int32),
          pltpu.VMEM((edges_per_tile,), jnp.int32),
          pltpu.VMEM((CHUNK, d), jnp.float32),
          pltpu.VMEM((CHUNK, d), jnp.float32),
          pltpu.VMEM_SHARED((rows_alloc, d), jnp.float32),
          pltpu.SemaphoreType.DMA,
          pltpu.SemaphoreType.DMA,
      ],
  )
  def scatter_add(h_hbm, src_hbm, dst_hbm, out_hbm,
                  sidx, didx, rows_a, rows_b, agg, sem_a, sem_b):
    c = lax.axis_index("c")
    s = lax.axis_index("s")
    wid = c * NS + s
    pltpu.sync_copy(src_hbm.at[pl.ds(wid * edges_per_tile, edges_per_tile)],
                    sidx)
    pltpu.sync_copy(dst_hbm.at[pl.ds(wid * edges_per_tile, edges_per_tile)],
                    didx)
    zvec = jnp.zeros((16,), jnp.float32)

    def zstore(t, carry):
      rows_a[t // (d // 16), pl.ds((t % (d // 16)) * 16, 16)] = zvec
      return carry

    lax.fori_loop(0, CHUNK * (d // 16), zstore, 0)
    for j in range(nz):
      pltpu.sync_copy(rows_a.at[pl.ds(0, zchunk)],
                      agg.at[pl.ds(s * zrpt + j * zchunk, zchunk)])
    plsc.subcore_barrier()

    ga = pltpu.async_copy(h_hbm.at[sidx.at[pl.ds(0, CHUNK)]], rows_a, sem_a)
    if n_chunks > 1:
      gb = pltpu.async_copy(
          h_hbm.at[sidx.at[pl.ds(CHUNK, CHUNK)]], rows_b, sem_b)

    def body(k, carry):
      i = 2 * k
      ga.wait()
      pltpu.sync_copy(rows_a, agg.at[didx.at[pl.ds(i * CHUNK, CHUNK)]],
                      add=True)

      @pl.when(i + 2 < n_chunks)
      def _():
        pltpu.async_copy(
            h_hbm.at[sidx.at[pl.ds((i + 2) * CHUNK, CHUNK)]], rows_a, sem_a)

      gb.wait()
      pltpu.sync_copy(rows_b, agg.at[didx.at[pl.ds((i + 1) * CHUNK, CHUNK)]],
                      add=True)

      @pl.when(i + 3 < n_chunks)
      def _():
        pltpu.async_copy(
            h_hbm.at[sidx.at[pl.ds((i + 3) * CHUNK, CHUNK)]], rows_b, sem_b)

      return carry

    lax.fori_loop(0, npairs, body, 0)
    if rem:
      ga.wait()
      pltpu.sync_copy(
          rows_a, agg.at[didx.at[pl.ds((n_chunks - 1) * CHUNK, CHUNK)]],
          add=True)
    plsc.subcore_barrier()
    pltpu.sync_copy(agg.at[pl.ds(s * zrpt, zrpt)],
                    out_hbm.at[c, pl.ds(s * zrpt, zrpt)])

  return scatter_add


def _mlp_body(scale_ref, x_ref, a0_ref, a1_ref, wa_ref, ba_ref, wb_ref,
              bb_ref, o_ref):
  h = x_ref[...] * scale_ref[0] + a0_ref[...] + a1_ref[...]
  h = jnp.dot(h, wa_ref[...], preferred_element_type=jnp.float32) + ba_ref[...]
  h = jnp.maximum(h, 0.0)
  h = jnp.dot(h, wb_ref[...], preferred_element_type=jnp.float32) + bb_ref[...]
  o_ref[...] = jnp.maximum(h, 0.0)


def _mlp(scale, x, a0, a1, Wa, ba, Wb, bb, block=1024):
  n, d = x.shape
  dh = Wa.shape[1]
  do = Wb.shape[1]
  return pl.pallas_call(
      _mlp_body,
      grid=(n // block,),
      in_specs=[
          pl.BlockSpec(memory_space=pltpu.SMEM),
          pl.BlockSpec((block, d), lambda i: (i, 0)),
          pl.BlockSpec((block, d), lambda i: (i, 0)),
          pl.BlockSpec((block, d), lambda i: (i, 0)),
          pl.BlockSpec((d, dh), lambda i: (0, 0)),
          pl.BlockSpec((1, dh), lambda i: (0, 0)),
          pl.BlockSpec((dh, do), lambda i: (0, 0)),
          pl.BlockSpec((1, do), lambda i: (0, 0)),
      ],
      out_specs=pl.BlockSpec((block, do), lambda i: (i, 0)),
      out_shape=jax.ShapeDtypeStruct((n, do), jnp.float32),
  )(scale, x, a0, a1, Wa, ba, Wb, bb)


def _final_body(p0_ref, p1_ref, w1_ref, b1_ref, w2_ref, b2_ref, o_ref):
  p = p0_ref[...] + p1_ref[...]
  t = jnp.dot(p, w1_ref[...], preferred_element_type=jnp.float32) + b1_ref[...]
  t = jnp.maximum(t, 0.0)
  o_ref[...] = (
      jnp.dot(t, w2_ref[...], preferred_element_type=jnp.float32) + b2_ref[...])


def kernel(x, edge_index, batch, eps0, W0a, b0a, W0b, b0b,
           eps1, W1a, b1a, W1b, b1b, eps2, W2a, b2a, W2b, b2b,
           Wf1, bf1, Wf2, bf2):
  n, d = x.shape
  e = edge_index.shape[1]
  g = 128
  edge_alloc = NS * 128 * (-(-n // (NS * 128)))
  pool_alloc = 2 * NS * 8
  tile_quant = NC * NS * CHUNK

  e_pad = ((e + tile_quant - 1) // tile_quant) * tile_quant
  assert e_pad == e or edge_alloc > n
  pad_ids = jnp.arange(e_pad - e, dtype=jnp.int32)
  src = jnp.concatenate([edge_index[0], pad_ids % n])
  dst = jnp.concatenate([edge_index[1], n + pad_ids % (edge_alloc - n)])

  n_pad = ((n + tile_quant - 1) // tile_quant) * tile_quant
  pool_pad = jnp.arange(n_pad - n, dtype=jnp.int32)
  pool_src = jnp.concatenate(
      [jnp.arange(n, dtype=jnp.int32), pool_pad % n])
  pool_dst = jnp.concatenate(
      [batch.astype(jnp.int32), g + pool_pad % (pool_alloc - g)])

  edge_scat = _scatter_add_factory(e_pad, edge_alloc, CHUNK, d)
  pool_scat = _scatter_add_factory(n_pad, pool_alloc, pool_alloc // NS, d)

  ba0, bb0 = b0a.reshape(1, -1), b0b.reshape(1, -1)
  ba1, bb1 = b1a.reshape(1, -1), b1b.reshape(1, -1)
  ba2, bb2 = b2a.reshape(1, -1), b2b.reshape(1, -1)
  s0 = (1.0 + eps0).reshape(1)
  s1 = (1.0 + eps1).reshape(1)
  s2 = (1.0 + eps2).reshape(1)

  xp = jnp.pad(x, ((0, edge_alloc - n), (0, 0)))
  a = edge_scat(xp, src, dst)
  h = _mlp(s0, xp, a[0], a[1], W0a, ba0, W0b, bb0)
  a = edge_scat(h, src, dst)
  h = _mlp(s1, h, a[0], a[1], W1a, ba1, W1b, bb1)
  a = edge_scat(h, src, dst)
  h = _mlp(s2, h, a[0], a[1], W2a, ba2, W2b, bb2)

  p = pool_scat(h, pool_src, pool_dst)

  out = pl.pallas_call(
      _final_body,
      out_shape=jax.ShapeDtypeStruct((g, 1), jnp.float32),
  )(p[0, :g], p[1, :g], Wf1, bf1.reshape(1, -1), Wf2, bf2.reshape(1, -1))
  return out
